# Initial kernel scaffold; baseline (speedup 1.0000x reference)
import jax
import jax.numpy as jnp
from jax.experimental import pallas as pl


def kernel(feature, edge_index_f, edge_index_b, W_f, attn_l_f, attn_r_f, bias_f, W_b, attn_l_b, attn_r_b, bias_b):
    raise NotImplementedError("write your pallas kernel here")


# speedup vs baseline: 20.4900x; 20.4900x over previous
import functools

import jax
import jax.numpy as jnp
from jax import lax
from jax.experimental import pallas as pl
from jax.experimental.pallas import tpu as pltpu
from jax.experimental.pallas import tpu_sc as plsc

N = 10000
E = 320000
D = 128
H = 4
NEG = 0.2

NP = 10240
NT = 16
PT = NP // NT
ET = E // NT
K = 80
NB = ET // K
KG = K // 16


def _stage1_body(x_ref, wf_ref, wb_ref, alf_ref, arf_ref, alb_ref, arb_ref,
                 hmat_ref, el_ref, er_ref):
    x = x_ref[...]
    for d_i, (w_ref, al_ref, ar_ref) in enumerate(
            ((wf_ref, alf_ref, arf_ref), (wb_ref, alb_ref, arb_ref))):
        for h_i in range(H):
            w = w_ref[:, h_i * D:(h_i + 1) * D]
            hh = jnp.dot(x, w, preferred_element_type=jnp.float32,
                         precision=lax.Precision.HIGHEST)
            k = d_i * H + h_i
            hmat_ref[k] = hh
            al = al_ref[h_i, :]
            ar = ar_ref[h_i, :]
            el_ref[k:k + 1, :] = jnp.sum(hh * al[None, :], axis=1)[None, :]
            er_ref[k:k + 1, :] = jnp.sum(hh * ar[None, :], axis=1)[None, :]


def _stage1(feature, W_f, W_b, alf, arf, alb, arb):
    BLK = 512
    xp = jnp.pad(feature, ((0, NP - N), (0, 0)))
    pad_a = lambda a: jnp.pad(a, ((0, 8 - H), (0, 0)))
    return pl.pallas_call(
        _stage1_body,
        grid=(NP // BLK,),
        in_specs=[
            pl.BlockSpec((BLK, D), lambda i: (i, 0)),
            pl.BlockSpec((D, H * D), lambda i: (0, 0)),
            pl.BlockSpec((D, H * D), lambda i: (0, 0)),
            pl.BlockSpec((8, D), lambda i: (0, 0)),
            pl.BlockSpec((8, D), lambda i: (0, 0)),
            pl.BlockSpec((8, D), lambda i: (0, 0)),
            pl.BlockSpec((8, D), lambda i: (0, 0)),
        ],
        out_specs=[
            pl.BlockSpec((8, BLK, D), lambda i: (0, i, 0)),
            pl.BlockSpec((8, BLK), lambda i: (0, i)),
            pl.BlockSpec((8, BLK), lambda i: (0, i)),
        ],
        out_shape=[
            jax.ShapeDtypeStruct((8, NP, D), jnp.float32),
            jax.ShapeDtypeStruct((8, NP), jnp.float32),
            jax.ShapeDtypeStruct((8, NP), jnp.float32),
        ],
    )(xp, W_f, W_b, pad_a(alf), pad_a(arf), pad_a(alb), pad_a(arb))


def _mbound_body(el_ref, er_ref, m_ref):
    mel = jnp.max(el_ref[...], axis=1)
    mer = jnp.max(er_ref[...], axis=1)
    m = jnp.maximum(mel + mer, 0.0)
    m_ref[...] = jnp.broadcast_to(m[:, None], (8, 128))


def _mbound(el, er):
    return pl.pallas_call(
        _mbound_body,
        grid=(1,),
        in_specs=[
            pl.BlockSpec((8, NP), lambda i: (0, 0)),
            pl.BlockSpec((8, NP), lambda i: (0, 0)),
        ],
        out_specs=pl.BlockSpec((8, 128), lambda i: (0, 0)),
        out_shape=jax.ShapeDtypeStruct((8, 128), jnp.float32),
    )(el, er)


def _sc_body(hflat, el_h, er_h, mflat, ef_s, ef_d, eb_s, eb_d, rst, den,
             acc_h, den_v, el_v, er_v, srcb, dstb, idx2b, exbuf,
             rows_v, z128, m_v, sem):
    core = lax.axis_index("c")
    tid = lax.axis_index("s")
    zero16 = jnp.zeros((16,), jnp.float32)
    iota16 = lax.iota(jnp.int32, 16)

    def _z128(i, c):
        for c2 in range(8):
            z128[i, pl.ds(c2 * 16, 16)] = zero16
        return c
    lax.fori_loop(0, 32, _z128, 0)

    lane0 = iota16 == 0

    base_n = tid * PT

    for d_i in range(2):
        e_src, e_dst = (ef_s, ef_d) if d_i == 0 else (eb_s, eb_d)
        for j in range(2):
            k_dyn = d_i * H + core * 2 + j

            pltpu.sync_copy(el_h.at[k_dyn], el_v)
            pltpu.sync_copy(er_h.at[k_dyn], er_v)

            pltpu.sync_copy(mflat.at[pl.ds(k_dyn * 128, 16)], m_v)
            m_vec = m_v[pl.ds(0, 16)]

            for z in range(20):
                pltpu.sync_copy(z128, acc_h.at[pl.ds(base_n + z * 32, 32)])

            def _zden(i, c):
                den_v[pl.ds(i * 16, 16)] = zero16
                return c
            lax.fori_loop(0, NP // 16, _zden, 0)
            plsc.subcore_barrier()

            k_off = k_dyn * NP

            def _batch(b, c):
                base_e = tid * ET + b * K
                pltpu.sync_copy(e_src.at[pl.ds(base_e, K)], srcb)
                pltpu.sync_copy(e_dst.at[pl.ds(base_e, K)], dstb)

                def _ex(g, c2):
                    src = srcb[pl.ds(g * 16, 16)]
                    dst = dstb[pl.ds(g * 16, 16)]
                    a = plsc.load_gather(el_v, [src])
                    r = plsc.load_gather(er_v, [dst])
                    s = a + r
                    s = jnp.where(s > 0, s, s * NEG)
                    exv = jnp.exp(s - m_vec)
                    exbuf[pl.ds(g * 16, 16)] = exv
                    idx2b[pl.ds(g * 16, 16)] = src + k_off
                    return c2
                lax.fori_loop(0, KG, _ex, 0)

                pltpu.async_copy(hflat.at[idx2b], rows_v, sem).wait()

                def _scale(e2, c2):
                    e_splat = jnp.full((16,), e2, jnp.int32)
                    sc = plsc.load_gather(exbuf, [e_splat])
                    dsti = plsc.load_gather(dstb, [e_splat])
                    plsc.addupdate_scatter(den_v, [dsti], sc, mask=lane0)
                    for c3 in range(8):
                        rows_v[e2, pl.ds(c3 * 16, 16)] = (
                            rows_v[e2, pl.ds(c3 * 16, 16)] * sc)
                    return c2
                lax.fori_loop(0, K, _scale, 0)

                pltpu.sync_copy(rows_v, acc_h.at[dstb], add=True)
                return c
            lax.fori_loop(0, NB, _batch, 0)
            plsc.subcore_barrier()

            pltpu.sync_copy(acc_h.at[pl.ds(base_n, PT)],
                            rst.at[k_dyn, pl.ds(base_n, PT)])
            pltpu.sync_copy(den_v, den.at[k_dyn, tid])


def _stage2(hmat, el, er, mrep, edge_index_f, edge_index_b):
    hflat = hmat.reshape(8 * NP, D)
    mflat = mrep.reshape(8 * 128)
    mesh = plsc.VectorSubcoreMesh(core_axis_name="c", subcore_axis_name="s")
    fn = pl.kernel(
        _sc_body,
        out_type=(
            jax.ShapeDtypeStruct((8, NP, D), jnp.float32),
            jax.ShapeDtypeStruct((8, NT, NP), jnp.float32),
        ),
        mesh=mesh,
        compiler_params=pltpu.CompilerParams(needs_layout_passes=False),
        scratch_types=[
            pltpu.VMEM_SHARED((NP, D), jnp.float32),
            pltpu.VMEM((NP,), jnp.float32),
            pltpu.VMEM((NP,), jnp.float32),
            pltpu.VMEM((NP,), jnp.float32),
            pltpu.VMEM((K,), jnp.int32),
            pltpu.VMEM((K,), jnp.int32),
            pltpu.VMEM((K,), jnp.int32),
            pltpu.VMEM((K,), jnp.float32),
            pltpu.VMEM((K, D), jnp.float32),
            pltpu.VMEM((32, D), jnp.float32),
            pltpu.VMEM((16,), jnp.float32),
            pltpu.SemaphoreType.DMA,
        ],
    )
    return fn(hflat, el, er, mflat,
              edge_index_f[0], edge_index_f[1],
              edge_index_b[0], edge_index_b[1])


def _combine_body(rst_ref, den_ref, out_ref):
    acc = None
    for k in range(8):
        num = rst_ref[k]
        dn = jnp.sum(den_ref[k], axis=0)
        dn = jnp.where(dn == 0.0, 1.0, dn)
        term = num / dn[:, None]
        acc = term if acc is None else acc + term
    out_ref[...] = acc * (1.0 / H)


def _stage3(rst, den):
    BLK = 512
    return pl.pallas_call(
        _combine_body,
        grid=(NP // BLK,),
        in_specs=[
            pl.BlockSpec((8, BLK, D), lambda i: (0, i, 0)),
            pl.BlockSpec((8, NT, BLK), lambda i: (0, 0, i)),
        ],
        out_specs=pl.BlockSpec((BLK, D), lambda i: (i, 0)),
        out_shape=jax.ShapeDtypeStruct((NP, D), jnp.float32),
    )(rst, den)


def kernel(feature, edge_index_f, edge_index_b, W_f, attn_l_f, attn_r_f,
           bias_f, W_b, attn_l_b, attn_r_b, bias_b):
    hmat, el, er = _stage1(feature, W_f, W_b, attn_l_f, attn_r_f,
                           attn_l_b, attn_r_b)
    mrep = _mbound(el, er)
    rst, den = _stage2(hmat, el, er, mrep, edge_index_f, edge_index_b)
    out = _stage3(rst, den)[:N]
    bias = jnp.mean(bias_f, axis=0) + jnp.mean(bias_b, axis=0)
    return out + bias[None, :]


# speedup vs baseline: 28.1513x; 1.3739x over previous
import functools

import jax
import jax.numpy as jnp
from jax import lax
from jax.experimental import pallas as pl
from jax.experimental.pallas import tpu as pltpu
from jax.experimental.pallas import tpu_sc as plsc

N = 10000
E = 320000
D = 128
H = 4
NEG = 0.2

NP = 10240
NT = 16
PT = NP // NT
ET = E // NT
K = 80
NB = ET // K
KG = K // 16


def _stage1_body(x_ref, wf_ref, wb_ref, alf_ref, arf_ref, alb_ref, arb_ref,
                 hmat_ref, el_ref, er_ref):
    x = x_ref[...]
    for d_i, (w_ref, al_ref, ar_ref) in enumerate(
            ((wf_ref, alf_ref, arf_ref), (wb_ref, alb_ref, arb_ref))):
        for h_i in range(H):
            w = w_ref[:, h_i * D:(h_i + 1) * D]
            hh = jnp.dot(x, w, preferred_element_type=jnp.float32,
                         precision=lax.Precision.HIGHEST)
            k = d_i * H + h_i
            hmat_ref[k] = hh
            al = al_ref[h_i, :]
            ar = ar_ref[h_i, :]
            el_ref[k:k + 1, :] = jnp.sum(hh * al[None, :], axis=1)[None, :]
            er_ref[k:k + 1, :] = jnp.sum(hh * ar[None, :], axis=1)[None, :]


def _stage1(feature, W_f, W_b, alf, arf, alb, arb):
    BLK = 512
    xp = jnp.pad(feature, ((0, NP - N), (0, 0)))
    pad_a = lambda a: jnp.pad(a, ((0, 8 - H), (0, 0)))
    return pl.pallas_call(
        _stage1_body,
        grid=(NP // BLK,),
        in_specs=[
            pl.BlockSpec((BLK, D), lambda i: (i, 0)),
            pl.BlockSpec((D, H * D), lambda i: (0, 0)),
            pl.BlockSpec((D, H * D), lambda i: (0, 0)),
            pl.BlockSpec((8, D), lambda i: (0, 0)),
            pl.BlockSpec((8, D), lambda i: (0, 0)),
            pl.BlockSpec((8, D), lambda i: (0, 0)),
            pl.BlockSpec((8, D), lambda i: (0, 0)),
        ],
        out_specs=[
            pl.BlockSpec((8, BLK, D), lambda i: (0, i, 0)),
            pl.BlockSpec((8, BLK), lambda i: (0, i)),
            pl.BlockSpec((8, BLK), lambda i: (0, i)),
        ],
        out_shape=[
            jax.ShapeDtypeStruct((8, NP, D), jnp.float32),
            jax.ShapeDtypeStruct((8, NP), jnp.float32),
            jax.ShapeDtypeStruct((8, NP), jnp.float32),
        ],
    )(xp, W_f, W_b, pad_a(alf), pad_a(arf), pad_a(alb), pad_a(arb))


def _mbound_body(el_ref, er_ref, m_ref):
    mel = jnp.max(el_ref[...], axis=1)
    mer = jnp.max(er_ref[...], axis=1)
    m = jnp.maximum(mel + mer, 0.0)
    m_ref[...] = jnp.broadcast_to(m[:, None], (8, 128))


def _mbound(el, er):
    return pl.pallas_call(
        _mbound_body,
        grid=(1,),
        in_specs=[
            pl.BlockSpec((8, NP), lambda i: (0, 0)),
            pl.BlockSpec((8, NP), lambda i: (0, 0)),
        ],
        out_specs=pl.BlockSpec((8, 128), lambda i: (0, 0)),
        out_shape=jax.ShapeDtypeStruct((8, 128), jnp.float32),
    )(el, er)


def _sc_body(hflat, el_h, er_h, mflat, ef_s, ef_d, eb_s, eb_d, rst, den,
             acc_h, den_v, el_v, er_v, srcb, dstb, idx2b, exbuf,
             rows_v, z128, m_v, semE1, semE2, semG1, semG2, semS):
    core = lax.axis_index("c")
    tid = lax.axis_index("s")
    zero16 = jnp.zeros((16,), jnp.float32)
    iota16 = lax.iota(jnp.int32, 16)

    def _z128(i, c):
        for c2 in range(8):
            z128[i, pl.ds(c2 * 16, 16)] = zero16
        return c
    lax.fori_loop(0, 32, _z128, 0)

    lane0 = iota16 == 0

    base_n = tid * PT

    for d_i in range(2):
        e_src, e_dst = (ef_s, ef_d) if d_i == 0 else (eb_s, eb_d)
        for j in range(2):
            k_dyn = d_i * H + core * 2 + j

            pltpu.sync_copy(el_h.at[k_dyn], el_v)
            pltpu.sync_copy(er_h.at[k_dyn], er_v)

            pltpu.sync_copy(mflat.at[pl.ds(k_dyn * 128, 16)], m_v)
            m_vec = m_v[pl.ds(0, 16)]

            for z in range(20):
                pltpu.sync_copy(z128, acc_h.at[pl.ds(base_n + z * 32, 32)])

            def _zden(i, c):
                den_v[pl.ds(i * 16, 16)] = zero16
                return c
            lax.fori_loop(0, NP // 16, _zden, 0)
            plsc.subcore_barrier()

            k_off = k_dyn * NP
            base_t = tid * ET
            max_base = E - K

            def _prefetch(b_next, pn):
                base = jnp.minimum(base_t + b_next * K, max_base)
                pltpu.async_copy(e_src.at[pl.ds(base, K)],
                                 srcb.at[pn], semE1)
                pltpu.async_copy(e_dst.at[pl.ds(base, K)],
                                 dstb.at[pn], semE2)

            def _do_batch(b, p, first):
                sb = srcb.at[p]
                db = dstb.at[p]
                pltpu.make_async_copy(e_src.at[pl.ds(0, K)], sb, semE1).wait()
                pltpu.make_async_copy(e_dst.at[pl.ds(0, K)], db, semE2).wait()

                def _ex(g, c2):
                    src = sb[pl.ds(g * 16, 16)]
                    dst = db[pl.ds(g * 16, 16)]
                    a = plsc.load_gather(el_v, [src])
                    r = plsc.load_gather(er_v, [dst])
                    s = a + r
                    s = jnp.where(s > 0, s, s * NEG)
                    exv = jnp.exp(s - m_vec)
                    exbuf[pl.ds(g * 16, 16)] = exv
                    idx2b[pl.ds(g * 16, 16)] = src + k_off
                    return c2
                lax.fori_loop(0, KG, _ex, 0)

                if first is None:
                    pltpu.make_async_copy(
                        rows_v, acc_h.at[db], semS).wait()
                else:
                    @pl.when(first)
                    def _():
                        pltpu.make_async_copy(
                            rows_v, acc_h.at[db], semS).wait()

                _prefetch(b + 1, 1 - p)

                pltpu.async_copy(hflat.at[idx2b.at[pl.ds(0, 48)]],
                                 rows_v.at[pl.ds(0, 48)], semG1)
                pltpu.async_copy(hflat.at[idx2b.at[pl.ds(48, 32)]],
                                 rows_v.at[pl.ds(48, 32)], semG2)

                def _scale(e2, c2):
                    e_splat = jnp.full((16,), e2, jnp.int32)
                    sc = plsc.load_gather(exbuf, [e_splat])
                    dsti = plsc.load_gather(db, [e_splat])
                    plsc.addupdate_scatter(den_v, [dsti], sc, mask=lane0)
                    for c3 in range(8):
                        rows_v[e2, pl.ds(c3 * 16, 16)] = (
                            rows_v[e2, pl.ds(c3 * 16, 16)] * sc)
                    return c2

                pltpu.make_async_copy(hflat.at[idx2b.at[pl.ds(0, 48)]],
                                      rows_v.at[pl.ds(0, 48)], semG1).wait()
                lax.fori_loop(0, 48, _scale, 0)
                pltpu.make_async_copy(hflat.at[idx2b.at[pl.ds(48, 32)]],
                                      rows_v.at[pl.ds(48, 32)], semG2).wait()
                lax.fori_loop(48, K, _scale, 0)

                pltpu.async_copy(rows_v, acc_h.at[db], semS, add=True)

            _prefetch(jnp.int32(0), 0)

            def _pair(b2, c):
                _do_batch(b2 * 2, 0, b2 > 0)
                _do_batch(b2 * 2 + 1, 1, None)
                return c
            lax.fori_loop(0, NB // 2, _pair, 0)

            pltpu.make_async_copy(rows_v, acc_h.at[dstb.at[1]], semS).wait()
            pltpu.make_async_copy(e_src.at[pl.ds(0, K)], srcb.at[0],
                                  semE1).wait()
            pltpu.make_async_copy(e_dst.at[pl.ds(0, K)], dstb.at[0],
                                  semE2).wait()
            plsc.subcore_barrier()

            pltpu.sync_copy(acc_h.at[pl.ds(base_n, PT)],
                            rst.at[k_dyn, pl.ds(base_n, PT)])
            pltpu.sync_copy(den_v, den.at[k_dyn, tid])


def _stage2(hmat, el, er, mrep, edge_index_f, edge_index_b):
    hflat = hmat.reshape(8 * NP, D)
    mflat = mrep.reshape(8 * 128)
    mesh = plsc.VectorSubcoreMesh(core_axis_name="c", subcore_axis_name="s")
    fn = pl.kernel(
        _sc_body,
        out_type=(
            jax.ShapeDtypeStruct((8, NP, D), jnp.float32),
            jax.ShapeDtypeStruct((8, NT, NP), jnp.float32),
        ),
        mesh=mesh,
        compiler_params=pltpu.CompilerParams(needs_layout_passes=False),
        scratch_types=[
            pltpu.VMEM_SHARED((NP, D), jnp.float32),
            pltpu.VMEM((NP,), jnp.float32),
            pltpu.VMEM((NP,), jnp.float32),
            pltpu.VMEM((NP,), jnp.float32),
            pltpu.VMEM((2, K), jnp.int32),
            pltpu.VMEM((2, K), jnp.int32),
            pltpu.VMEM((K,), jnp.int32),
            pltpu.VMEM((K,), jnp.float32),
            pltpu.VMEM((K, D), jnp.float32),
            pltpu.VMEM((32, D), jnp.float32),
            pltpu.VMEM((16,), jnp.float32),
            pltpu.SemaphoreType.DMA,
            pltpu.SemaphoreType.DMA,
            pltpu.SemaphoreType.DMA,
            pltpu.SemaphoreType.DMA,
            pltpu.SemaphoreType.DMA,
        ],
    )
    return fn(hflat, el, er, mflat,
              edge_index_f[0], edge_index_f[1],
              edge_index_b[0], edge_index_b[1])


def _combine_body(rst_ref, den_ref, out_ref):
    acc = None
    for k in range(8):
        num = rst_ref[k]
        dn = jnp.sum(den_ref[k], axis=0)
        dn = jnp.where(dn == 0.0, 1.0, dn)
        term = num / dn[:, None]
        acc = term if acc is None else acc + term
    out_ref[...] = acc * (1.0 / H)


def _stage3(rst, den):
    BLK = 512
    return pl.pallas_call(
        _combine_body,
        grid=(NP // BLK,),
        in_specs=[
            pl.BlockSpec((8, BLK, D), lambda i: (0, i, 0)),
            pl.BlockSpec((8, NT, BLK), lambda i: (0, 0, i)),
        ],
        out_specs=pl.BlockSpec((BLK, D), lambda i: (i, 0)),
        out_shape=jax.ShapeDtypeStruct((NP, D), jnp.float32),
    )(rst, den)


def kernel(feature, edge_index_f, edge_index_b, W_f, attn_l_f, attn_r_f,
           bias_f, W_b, attn_l_b, attn_r_b, bias_b):
    hmat, el, er = _stage1(feature, W_f, W_b, attn_l_f, attn_r_f,
                           attn_l_b, attn_r_b)
    mrep = _mbound(el, er)
    rst, den = _stage2(hmat, el, er, mrep, edge_index_f, edge_index_b)
    out = _stage3(rst, den)[:N]
    bias = jnp.mean(bias_f, axis=0) + jnp.mean(bias_b, axis=0)
    return out + bias[None, :]


# speedup vs baseline: 28.9696x; 1.0291x over previous
import functools

import jax
import jax.numpy as jnp
from jax import lax
from jax.experimental import pallas as pl
from jax.experimental.pallas import tpu as pltpu
from jax.experimental.pallas import tpu_sc as plsc

N = 10000
E = 320000
D = 128
H = 4
NEG = 0.2

NP = 10240
NT = 16
PT = NP // NT
ET = E // NT
K = 80
NB = ET // K
KG = K // 16


def _stage1_body(x_ref, wf_ref, wb_ref, alf_ref, arf_ref, alb_ref, arb_ref,
                 hmat_ref, el_ref, er_ref):
    x = x_ref[...]
    for d_i, (w_ref, al_ref, ar_ref) in enumerate(
            ((wf_ref, alf_ref, arf_ref), (wb_ref, alb_ref, arb_ref))):
        for h_i in range(H):
            w = w_ref[:, h_i * D:(h_i + 1) * D]
            hh = jnp.dot(x, w, preferred_element_type=jnp.float32,
                         precision=lax.Precision.HIGHEST)
            k = d_i * H + h_i
            hmat_ref[k] = hh
            al = al_ref[h_i, :]
            ar = ar_ref[h_i, :]
            el_ref[k:k + 1, :] = jnp.sum(hh * al[None, :], axis=1)[None, :]
            er_ref[k:k + 1, :] = jnp.sum(hh * ar[None, :], axis=1)[None, :]


def _stage1(feature, W_f, W_b, alf, arf, alb, arb):
    BLK = 512
    xp = jnp.pad(feature, ((0, NP - N), (0, 0)))
    pad_a = lambda a: jnp.pad(a, ((0, 8 - H), (0, 0)))
    return pl.pallas_call(
        _stage1_body,
        grid=(NP // BLK,),
        in_specs=[
            pl.BlockSpec((BLK, D), lambda i: (i, 0)),
            pl.BlockSpec((D, H * D), lambda i: (0, 0)),
            pl.BlockSpec((D, H * D), lambda i: (0, 0)),
            pl.BlockSpec((8, D), lambda i: (0, 0)),
            pl.BlockSpec((8, D), lambda i: (0, 0)),
            pl.BlockSpec((8, D), lambda i: (0, 0)),
            pl.BlockSpec((8, D), lambda i: (0, 0)),
        ],
        out_specs=[
            pl.BlockSpec((8, BLK, D), lambda i: (0, i, 0)),
            pl.BlockSpec((8, BLK), lambda i: (0, i)),
            pl.BlockSpec((8, BLK), lambda i: (0, i)),
        ],
        out_shape=[
            jax.ShapeDtypeStruct((8, NP, D), jnp.float32),
            jax.ShapeDtypeStruct((8, NP), jnp.float32),
            jax.ShapeDtypeStruct((8, NP), jnp.float32),
        ],
    )(xp, W_f, W_b, pad_a(alf), pad_a(arf), pad_a(alb), pad_a(arb))


def _mbound_body(el_ref, er_ref, m_ref):
    mel = jnp.max(el_ref[...], axis=1)
    mer = jnp.max(er_ref[...], axis=1)
    m = jnp.maximum(mel + mer, 0.0)
    m_ref[...] = jnp.broadcast_to(m[:, None], (8, 128))


def _mbound(el, er):
    return pl.pallas_call(
        _mbound_body,
        grid=(1,),
        in_specs=[
            pl.BlockSpec((8, NP), lambda i: (0, 0)),
            pl.BlockSpec((8, NP), lambda i: (0, 0)),
        ],
        out_specs=pl.BlockSpec((8, 128), lambda i: (0, 0)),
        out_shape=jax.ShapeDtypeStruct((8, 128), jnp.float32),
    )(el, er)


def _sc_body(hflat, el_h, er_h, mflat, ef_s, ef_d, eb_s, eb_d, rst, den,
             acc_h, den_v, el_v, er_v, srcb, dstb, idx2b, exbuf,
             rows_v, z128, m_v, semE1, semE2, semG1, semG2, semS):
    core = lax.axis_index("c")
    tid = lax.axis_index("s")
    zero16 = jnp.zeros((16,), jnp.float32)
    iota16 = lax.iota(jnp.int32, 16)

    def _z128(i, c):
        for c2 in range(8):
            z128[i, pl.ds(c2 * 16, 16)] = zero16
        return c
    lax.fori_loop(0, 32, _z128, 0)

    lane0 = iota16 == 0

    base_n = tid * PT

    for d_i in range(2):
        e_src, e_dst = (ef_s, ef_d) if d_i == 0 else (eb_s, eb_d)
        for j in range(2):
            k_dyn = d_i * H + core * 2 + j

            pltpu.sync_copy(el_h.at[k_dyn], el_v)
            pltpu.sync_copy(er_h.at[k_dyn], er_v)

            pltpu.sync_copy(mflat.at[pl.ds(k_dyn * 128, 16)], m_v)
            m_vec = m_v[pl.ds(0, 16)]

            for z in range(20):
                pltpu.sync_copy(z128, acc_h.at[pl.ds(base_n + z * 32, 32)])

            def _zden(i, c):
                den_v[pl.ds(i * 16, 16)] = zero16
                return c
            lax.fori_loop(0, NP // 16, _zden, 0)
            plsc.subcore_barrier()

            k_off = k_dyn * NP
            base_t = tid * ET
            max_base = E - K

            def _prefetch(b_next, pn):
                base = jnp.minimum(base_t + b_next * K, max_base)
                pltpu.async_copy(e_src.at[pl.ds(base, K)],
                                 srcb.at[pn], semE1)
                pltpu.async_copy(e_dst.at[pl.ds(base, K)],
                                 dstb.at[pn], semE2)

            def _do_batch(b, p, first):
                sb = srcb.at[p]
                db = dstb.at[p]
                pltpu.make_async_copy(e_src.at[pl.ds(0, K)], sb, semE1).wait()
                pltpu.make_async_copy(e_dst.at[pl.ds(0, K)], db, semE2).wait()

                def _ex(g, c2):
                    src = sb[pl.ds(g * 16, 16)]
                    dst = db[pl.ds(g * 16, 16)]
                    a = plsc.load_gather(el_v, [src])
                    r = plsc.load_gather(er_v, [dst])
                    s = a + r
                    s = jnp.where(s > 0, s, s * NEG)
                    exv = jnp.exp(s - m_vec)
                    exbuf[pl.ds(g * 16, 16)] = exv
                    idx2b[pl.ds(g * 16, 16)] = src + k_off
                    return c2
                lax.fori_loop(0, KG, _ex, 0)

                if first is None:
                    pltpu.make_async_copy(
                        rows_v, acc_h.at[db], semS).wait()
                else:
                    @pl.when(first)
                    def _():
                        pltpu.make_async_copy(
                            rows_v, acc_h.at[db], semS).wait()

                _prefetch(b + 1, 1 - p)

                pltpu.async_copy(hflat.at[idx2b.at[pl.ds(0, 32)]],
                                 rows_v.at[pl.ds(0, 32)], semG1)
                pltpu.async_copy(hflat.at[idx2b.at[pl.ds(32, 48)]],
                                 rows_v.at[pl.ds(32, 48)], semG2)

                def _scale(e2, c2):
                    e_splat = jnp.full((16,), e2, jnp.int32)
                    sc = plsc.load_gather(exbuf, [e_splat])
                    dsti = plsc.load_gather(db, [e_splat])
                    plsc.addupdate_scatter(den_v, [dsti], sc, mask=lane0)
                    for c3 in range(8):
                        rows_v[e2, pl.ds(c3 * 16, 16)] = (
                            rows_v[e2, pl.ds(c3 * 16, 16)] * sc)
                    return c2

                pltpu.make_async_copy(hflat.at[idx2b.at[pl.ds(0, 32)]],
                                      rows_v.at[pl.ds(0, 32)], semG1).wait()
                lax.fori_loop(0, 32, _scale, 0)
                pltpu.make_async_copy(hflat.at[idx2b.at[pl.ds(32, 48)]],
                                      rows_v.at[pl.ds(32, 48)], semG2).wait()
                lax.fori_loop(32, K, _scale, 0)

                pltpu.async_copy(rows_v, acc_h.at[db], semS, add=True)

            _prefetch(jnp.int32(0), 0)

            def _pair(b2, c):
                _do_batch(b2 * 2, 0, b2 > 0)
                _do_batch(b2 * 2 + 1, 1, None)
                return c
            lax.fori_loop(0, NB // 2, _pair, 0)

            pltpu.make_async_copy(rows_v, acc_h.at[dstb.at[1]], semS).wait()
            pltpu.make_async_copy(e_src.at[pl.ds(0, K)], srcb.at[0],
                                  semE1).wait()
            pltpu.make_async_copy(e_dst.at[pl.ds(0, K)], dstb.at[0],
                                  semE2).wait()
            plsc.subcore_barrier()

            pltpu.sync_copy(acc_h.at[pl.ds(base_n, PT)],
                            rst.at[k_dyn, pl.ds(base_n, PT)])
            pltpu.sync_copy(den_v, den.at[k_dyn, tid])


def _stage2(hmat, el, er, mrep, edge_index_f, edge_index_b):
    hflat = hmat.reshape(8 * NP, D)
    mflat = mrep.reshape(8 * 128)
    mesh = plsc.VectorSubcoreMesh(core_axis_name="c", subcore_axis_name="s")
    fn = pl.kernel(
        _sc_body,
        out_type=(
            jax.ShapeDtypeStruct((8, NP, D), jnp.float32),
            jax.ShapeDtypeStruct((8, NT, NP), jnp.float32),
        ),
        mesh=mesh,
        compiler_params=pltpu.CompilerParams(needs_layout_passes=False),
        scratch_types=[
            pltpu.VMEM_SHARED((NP, D), jnp.float32),
            pltpu.VMEM((NP,), jnp.float32),
            pltpu.VMEM((NP,), jnp.float32),
            pltpu.VMEM((NP,), jnp.float32),
            pltpu.VMEM((2, K), jnp.int32),
            pltpu.VMEM((2, K), jnp.int32),
            pltpu.VMEM((K,), jnp.int32),
            pltpu.VMEM((K,), jnp.float32),
            pltpu.VMEM((K, D), jnp.float32),
            pltpu.VMEM((32, D), jnp.float32),
            pltpu.VMEM((16,), jnp.float32),
            pltpu.SemaphoreType.DMA,
            pltpu.SemaphoreType.DMA,
            pltpu.SemaphoreType.DMA,
            pltpu.SemaphoreType.DMA,
            pltpu.SemaphoreType.DMA,
        ],
    )
    return fn(hflat, el, er, mflat,
              edge_index_f[0], edge_index_f[1],
              edge_index_b[0], edge_index_b[1])


def _combine_body(rst_ref, den_ref, out_ref):
    acc = None
    for k in range(8):
        num = rst_ref[k]
        dn = jnp.sum(den_ref[k], axis=0)
        dn = jnp.where(dn == 0.0, 1.0, dn)
        term = num / dn[:, None]
        acc = term if acc is None else acc + term
    out_ref[...] = acc * (1.0 / H)


def _stage3(rst, den):
    BLK = 512
    return pl.pallas_call(
        _combine_body,
        grid=(NP // BLK,),
        in_specs=[
            pl.BlockSpec((8, BLK, D), lambda i: (0, i, 0)),
            pl.BlockSpec((8, NT, BLK), lambda i: (0, 0, i)),
        ],
        out_specs=pl.BlockSpec((BLK, D), lambda i: (i, 0)),
        out_shape=jax.ShapeDtypeStruct((NP, D), jnp.float32),
    )(rst, den)


def kernel(feature, edge_index_f, edge_index_b, W_f, attn_l_f, attn_r_f,
           bias_f, W_b, attn_l_b, attn_r_b, bias_b):
    hmat, el, er = _stage1(feature, W_f, W_b, attn_l_f, attn_r_f,
                           attn_l_b, attn_r_b)
    mrep = _mbound(el, er)
    rst, den = _stage2(hmat, el, er, mrep, edge_index_f, edge_index_b)
    out = _stage3(rst, den)[:N]
    bias = jnp.mean(bias_f, axis=0) + jnp.mean(bias_b, axis=0)
    return out + bias[None, :]


# speedup vs baseline: 29.8302x; 1.0297x over previous
import functools

import jax
import jax.numpy as jnp
from jax import lax
from jax.experimental import pallas as pl
from jax.experimental.pallas import tpu as pltpu
from jax.experimental.pallas import tpu_sc as plsc

N = 10000
E = 320000
D = 128
H = 4
NEG = 0.2

NP = 10240
NT = 16
PT = NP // NT
ET = E // NT
K = 80
NB = ET // K
KG = K // 16


def _stage1_body(x_ref, wf_ref, wb_ref, alf_ref, arf_ref, alb_ref, arb_ref,
                 hmat_ref, el_ref, er_ref):
    x = x_ref[...]
    for d_i, (w_ref, al_ref, ar_ref) in enumerate(
            ((wf_ref, alf_ref, arf_ref), (wb_ref, alb_ref, arb_ref))):
        for h_i in range(H):
            w = w_ref[:, h_i * D:(h_i + 1) * D]
            hh = jnp.dot(x, w, preferred_element_type=jnp.float32,
                         precision=lax.Precision.HIGHEST)
            k = d_i * H + h_i
            hmat_ref[k] = hh
            al = al_ref[h_i, :]
            ar = ar_ref[h_i, :]
            el_ref[k:k + 1, :] = jnp.sum(hh * al[None, :], axis=1)[None, :]
            er_ref[k:k + 1, :] = jnp.sum(hh * ar[None, :], axis=1)[None, :]


def _stage1(feature, W_f, W_b, alf, arf, alb, arb):
    BLK = 512
    xp = jnp.pad(feature, ((0, NP - N), (0, 0)))
    pad_a = lambda a: jnp.pad(a, ((0, 8 - H), (0, 0)))
    return pl.pallas_call(
        _stage1_body,
        grid=(NP // BLK,),
        in_specs=[
            pl.BlockSpec((BLK, D), lambda i: (i, 0)),
            pl.BlockSpec((D, H * D), lambda i: (0, 0)),
            pl.BlockSpec((D, H * D), lambda i: (0, 0)),
            pl.BlockSpec((8, D), lambda i: (0, 0)),
            pl.BlockSpec((8, D), lambda i: (0, 0)),
            pl.BlockSpec((8, D), lambda i: (0, 0)),
            pl.BlockSpec((8, D), lambda i: (0, 0)),
        ],
        out_specs=[
            pl.BlockSpec((8, BLK, D), lambda i: (0, i, 0)),
            pl.BlockSpec((8, BLK), lambda i: (0, i)),
            pl.BlockSpec((8, BLK), lambda i: (0, i)),
        ],
        out_shape=[
            jax.ShapeDtypeStruct((8, NP, D), jnp.float32),
            jax.ShapeDtypeStruct((8, NP), jnp.float32),
            jax.ShapeDtypeStruct((8, NP), jnp.float32),
        ],
    )(xp, W_f, W_b, pad_a(alf), pad_a(arf), pad_a(alb), pad_a(arb))


def _mbound_body(el_ref, er_ref, m_ref):
    mel = jnp.max(el_ref[...], axis=1)
    mer = jnp.max(er_ref[...], axis=1)
    m = jnp.maximum(mel + mer, 0.0)
    m_ref[...] = jnp.broadcast_to(m[:, None], (8, 128))


def _mbound(el, er):
    return pl.pallas_call(
        _mbound_body,
        grid=(1,),
        in_specs=[
            pl.BlockSpec((8, NP), lambda i: (0, 0)),
            pl.BlockSpec((8, NP), lambda i: (0, 0)),
        ],
        out_specs=pl.BlockSpec((8, 128), lambda i: (0, 0)),
        out_shape=jax.ShapeDtypeStruct((8, 128), jnp.float32),
    )(el, er)


def _sc_body(hflat, el_h, er_h, mflat, ef_s, ef_d, eb_s, eb_d, rst, den,
             acc_h, den_v, el_v, er_v, srcb, dstb, idx2b, exbuf,
             rows_v, z128, m_v, semE1, semE2, semG1, semG2, semS):
    core = lax.axis_index("c")
    tid = lax.axis_index("s")
    zero16 = jnp.zeros((16,), jnp.float32)
    iota16 = lax.iota(jnp.int32, 16)

    def _z128(i, c):
        for c2 in range(8):
            z128[i, pl.ds(c2 * 16, 16)] = zero16
        return c
    lax.fori_loop(0, 32, _z128, 0)

    lane0 = iota16 == 0

    base_n = tid * PT

    for d_i in range(2):
        e_src, e_dst = (ef_s, ef_d) if d_i == 0 else (eb_s, eb_d)
        for j in range(2):
            k_dyn = d_i * H + core * 2 + j

            pltpu.sync_copy(el_h.at[k_dyn], el_v)
            pltpu.sync_copy(er_h.at[k_dyn], er_v)

            pltpu.sync_copy(mflat.at[pl.ds(k_dyn * 128, 16)], m_v)
            m_vec = m_v[pl.ds(0, 16)]

            for z in range(20):
                pltpu.sync_copy(z128, acc_h.at[pl.ds(base_n + z * 32, 32)])

            def _zden(i, c):
                den_v[pl.ds(i * 16, 16)] = zero16
                return c
            lax.fori_loop(0, NP // 16, _zden, 0)
            plsc.subcore_barrier()

            k_off = k_dyn * NP
            base_t = tid * ET
            max_base = E - K

            def _prefetch(b_next, pn):
                base = jnp.minimum(base_t + b_next * K, max_base)
                pltpu.async_copy(e_src.at[pl.ds(base, K)],
                                 srcb.at[pn], semE1)
                pltpu.async_copy(e_dst.at[pl.ds(base, K)],
                                 dstb.at[pn], semE2)

            def _do_batch(b, p, first):
                sb = srcb.at[p]
                db = dstb.at[p]
                pltpu.make_async_copy(e_src.at[pl.ds(0, K)], sb, semE1).wait()
                pltpu.make_async_copy(e_dst.at[pl.ds(0, K)], db, semE2).wait()

                def _ex(g, c2):
                    src = sb[pl.ds(g * 16, 16)]
                    dst = db[pl.ds(g * 16, 16)]
                    a = plsc.load_gather(el_v, [src])
                    r = plsc.load_gather(er_v, [dst])
                    s = a + r
                    s = jnp.where(s > 0, s, s * NEG)
                    exv = jnp.exp(s - m_vec)
                    exbuf[pl.ds(g * 16, 16)] = exv
                    idx2b[pl.ds(g * 16, 16)] = src + k_off
                    return c2
                lax.fori_loop(0, KG, _ex, 0)

                if first is None:
                    pltpu.make_async_copy(
                        rows_v, acc_h.at[db], semS).wait()
                else:
                    @pl.when(first)
                    def _():
                        pltpu.make_async_copy(
                            rows_v, acc_h.at[db], semS).wait()

                _prefetch(b + 1, 1 - p)

                pltpu.async_copy(hflat.at[idx2b.at[pl.ds(0, 16)]],
                                 rows_v.at[pl.ds(0, 16)], semG1)
                pltpu.async_copy(hflat.at[idx2b.at[pl.ds(16, 64)]],
                                 rows_v.at[pl.ds(16, 64)], semG2)

                def _scale(e2, c2):
                    e_splat = jnp.full((16,), e2, jnp.int32)
                    sc = plsc.load_gather(exbuf, [e_splat])
                    dsti = plsc.load_gather(db, [e_splat])
                    plsc.addupdate_scatter(den_v, [dsti], sc, mask=lane0)
                    for c3 in range(8):
                        rows_v[e2, pl.ds(c3 * 16, 16)] = (
                            rows_v[e2, pl.ds(c3 * 16, 16)] * sc)
                    return c2

                pltpu.make_async_copy(hflat.at[idx2b.at[pl.ds(0, 16)]],
                                      rows_v.at[pl.ds(0, 16)], semG1).wait()
                lax.fori_loop(0, 16, _scale, 0)
                pltpu.make_async_copy(hflat.at[idx2b.at[pl.ds(16, 64)]],
                                      rows_v.at[pl.ds(16, 64)], semG2).wait()
                lax.fori_loop(16, K, _scale, 0)

                pltpu.async_copy(rows_v, acc_h.at[db], semS, add=True)

            _prefetch(jnp.int32(0), 0)

            def _pair(b2, c):
                _do_batch(b2 * 2, 0, b2 > 0)
                _do_batch(b2 * 2 + 1, 1, None)
                return c
            lax.fori_loop(0, NB // 2, _pair, 0)

            pltpu.make_async_copy(rows_v, acc_h.at[dstb.at[1]], semS).wait()
            pltpu.make_async_copy(e_src.at[pl.ds(0, K)], srcb.at[0],
                                  semE1).wait()
            pltpu.make_async_copy(e_dst.at[pl.ds(0, K)], dstb.at[0],
                                  semE2).wait()
            plsc.subcore_barrier()

            pltpu.sync_copy(acc_h.at[pl.ds(base_n, PT)],
                            rst.at[k_dyn, pl.ds(base_n, PT)])
            pltpu.sync_copy(den_v, den.at[k_dyn, tid])


def _stage2(hmat, el, er, mrep, edge_index_f, edge_index_b):
    hflat = hmat.reshape(8 * NP, D)
    mflat = mrep.reshape(8 * 128)
    mesh = plsc.VectorSubcoreMesh(core_axis_name="c", subcore_axis_name="s")
    fn = pl.kernel(
        _sc_body,
        out_type=(
            jax.ShapeDtypeStruct((8, NP, D), jnp.float32),
            jax.ShapeDtypeStruct((8, NT, NP), jnp.float32),
        ),
        mesh=mesh,
        compiler_params=pltpu.CompilerParams(needs_layout_passes=False),
        scratch_types=[
            pltpu.VMEM_SHARED((NP, D), jnp.float32),
            pltpu.VMEM((NP,), jnp.float32),
            pltpu.VMEM((NP,), jnp.float32),
            pltpu.VMEM((NP,), jnp.float32),
            pltpu.VMEM((2, K), jnp.int32),
            pltpu.VMEM((2, K), jnp.int32),
            pltpu.VMEM((K,), jnp.int32),
            pltpu.VMEM((K,), jnp.float32),
            pltpu.VMEM((K, D), jnp.float32),
            pltpu.VMEM((32, D), jnp.float32),
            pltpu.VMEM((16,), jnp.float32),
            pltpu.SemaphoreType.DMA,
            pltpu.SemaphoreType.DMA,
            pltpu.SemaphoreType.DMA,
            pltpu.SemaphoreType.DMA,
            pltpu.SemaphoreType.DMA,
        ],
    )
    return fn(hflat, el, er, mflat,
              edge_index_f[0], edge_index_f[1],
              edge_index_b[0], edge_index_b[1])


def _combine_body(rst_ref, den_ref, out_ref):
    acc = None
    for k in range(8):
        num = rst_ref[k]
        dn = jnp.sum(den_ref[k], axis=0)
        dn = jnp.where(dn == 0.0, 1.0, dn)
        term = num / dn[:, None]
        acc = term if acc is None else acc + term
    out_ref[...] = acc * (1.0 / H)


def _stage3(rst, den):
    BLK = 512
    return pl.pallas_call(
        _combine_body,
        grid=(NP // BLK,),
        in_specs=[
            pl.BlockSpec((8, BLK, D), lambda i: (0, i, 0)),
            pl.BlockSpec((8, NT, BLK), lambda i: (0, 0, i)),
        ],
        out_specs=pl.BlockSpec((BLK, D), lambda i: (i, 0)),
        out_shape=jax.ShapeDtypeStruct((NP, D), jnp.float32),
    )(rst, den)


def kernel(feature, edge_index_f, edge_index_b, W_f, attn_l_f, attn_r_f,
           bias_f, W_b, attn_l_b, attn_r_b, bias_b):
    hmat, el, er = _stage1(feature, W_f, W_b, attn_l_f, attn_r_f,
                           attn_l_b, attn_r_b)
    mrep = _mbound(el, er)
    rst, den = _stage2(hmat, el, er, mrep, edge_index_f, edge_index_b)
    out = _stage3(rst, den)[:N]
    bias = jnp.mean(bias_f, axis=0) + jnp.mean(bias_b, axis=0)
    return out + bias[None, :]
